# Initial kernel scaffold; baseline (speedup 1.0000x reference)
import jax
import jax.numpy as jnp
from jax.experimental import pallas as pl


def kernel(t, node_f, edge_index, xf, W_ode, b_ode, W_ne, b_ne, W_ee, b_ee, W_nd, b_nd):
    raise NotImplementedError("write your pallas kernel here")


# speedup vs baseline: 8.0437x; 8.0437x over previous
import functools

import jax
import jax.numpy as jnp
from jax import lax
from jax.experimental import pallas as pl
from jax.experimental.pallas import tpu as pltpu
from jax.experimental.pallas import tpu_sc as plsc

N = 100000
E = 1600000
MD = 4
HID = 16


_BLK = 2000
_NBLK = N // _BLK


def _encode_body(nf_ref, wne_ref, bne_ref, wee_ref, bee_ref, wode_ref,
                 bode_ref, wnd1_ref, a_ref, b_ref, outp_ref):
    nf = nf_ref[...]
    ne = jnp.dot(nf, wne_ref[...], preferred_element_type=jnp.float32)
    ne = jnp.maximum(ne + bne_ref[...], 0.0)
    wee = wee_ref[...]
    a_ref[...] = jnp.dot(ne, wee[:HID], preferred_element_type=jnp.float32)
    b_ref[...] = (jnp.dot(ne, wee[HID:], preferred_element_type=jnp.float32)
                  + bee_ref[...])
    node_f = nf[:, 2:2 + MD]
    base = jnp.dot(node_f, wode_ref[...], preferred_element_type=jnp.float32)
    vel = jnp.dot(ne, wnd1_ref[...], preferred_element_type=jnp.float32)
    outp_ref[...] = base + bode_ref[...] + vel


_encode = pl.pallas_call(
    _encode_body,
    grid=(_NBLK,),
    in_specs=[
        pl.BlockSpec((_BLK, 2 + MD), lambda i: (i, 0)),
        pl.BlockSpec((2 + MD, HID), lambda i: (0, 0)),
        pl.BlockSpec((1, HID), lambda i: (0, 0)),
        pl.BlockSpec((2 * HID, HID), lambda i: (0, 0)),
        pl.BlockSpec((1, HID), lambda i: (0, 0)),
        pl.BlockSpec((MD, MD), lambda i: (0, 0)),
        pl.BlockSpec((1, MD), lambda i: (0, 0)),
        pl.BlockSpec((HID, MD), lambda i: (0, 0)),
    ],
    out_specs=[
        pl.BlockSpec((_BLK, HID), lambda i: (i, 0)),
        pl.BlockSpec((_BLK, HID), lambda i: (i, 0)),
        pl.BlockSpec((_BLK, MD), lambda i: (i, 0)),
    ],
    out_shape=[
        jax.ShapeDtypeStruct((N, HID), jnp.float32),
        jax.ShapeDtypeStruct((N, HID), jnp.float32),
        jax.ShapeDtypeStruct((N, MD), jnp.float32),
    ],
)


_NC = 2
_NS = 16
_NW = _NC * _NS
_IB = 128
_KB = 8
_EPC = _IB * _KB
_NROW = E // _IB
_NCH = _NROW // _KB
_TKB = _NROW - _NCH * _KB
_CH_PER_W = (_NCH + _NW - 1) // _NW
_HR = N // 2
_RPTA = 3136
_RPTL = _HR - (_NS - 1) * _RPTA


def _sc_body(src_hbm, dst_hbm, a_hbm, b_hbm, out_hbm,
             src_i, dst_i, dst2_i, a_rows, b_rows, acc, sem):
    c = lax.axis_index("c")
    s = lax.axis_index("s")
    wid = c * _NS + s
    r0 = s * _RPTA

    def _do_chunk(row0, kb, lo):
        pltpu.sync_copy(src_hbm.at[pl.ds(row0, kb)], src_i.at[pl.ds(0, kb)])
        pltpu.sync_copy(dst_hbm.at[pl.ds(row0, kb)], dst_i.at[pl.ds(0, kb)])
        cps = []
        for j in range(kb):
            cps.append(pltpu.async_copy(
                a_hbm.at[src_i.at[j]],
                a_rows.at[pl.ds(j * _IB, _IB)], sem))
            cps.append(pltpu.async_copy(
                b_hbm.at[dst_i.at[j]],
                b_rows.at[pl.ds(j * _IB, _IB)], sem))
        for cp in cps:
            cp.wait()

        for j in range(kb):
            for k in range(_IB // 16):
                d = dst_i[j, pl.ds(k * 16, 16)] - lo
                dummy = _HR + (k % 8) * 16 + lax.iota(jnp.int32, 16)
                inr = (d >= 0) & (d < _HR)
                dst2_i[j, pl.ds(k * 16, 16)] = jnp.where(inr, d, dummy)

        def _edge(e, carry2):
            a_rows[e] = jnp.maximum(a_rows[e] + b_rows[e], 0.0)
            return carry2

        lax.fori_loop(0, kb * _IB, _edge, 0, unroll=4)

        for j in range(kb):
            pltpu.sync_copy(a_rows.at[pl.ds(j * _IB, _IB)],
                            acc.at[dst2_i.at[j]], add=True)

    def _zero_fill(i, carry):
        b_rows[i] = jnp.zeros((HID,), jnp.float32)
        return carry

    def _zero_slice(cnt):
        nfull = cnt // _EPC
        for k in range(nfull):
            pltpu.sync_copy(b_rows.at[...],
                            acc.at[pl.ds(r0 + k * _EPC, _EPC)])
        rem = cnt - nfull * _EPC
        if rem:
            pltpu.sync_copy(b_rows.at[pl.ds(0, rem)],
                            acc.at[pl.ds(r0 + nfull * _EPC, rem)])

    for p in range(2):
        lax.fori_loop(0, _EPC, _zero_fill, 0)

        @pl.when(s < _NS - 1)
        def _():
            _zero_slice(_RPTA)

        @pl.when(s == _NS - 1)
        def _():
            _zero_slice(_RPTL)

        plsc.subcore_barrier()

        def _chunk(i, carry, p=p):
            ch = i * _NW + wid

            @pl.when(ch < _NCH)
            def _():
                _do_chunk(ch * _KB, _KB, p * _HR)

            return carry

        lax.fori_loop(0, _CH_PER_W, _chunk, 0)

        if _TKB:
            @pl.when(wid == 0)
            def _tail():
                _do_chunk(_NCH * _KB, _TKB, p * _HR)

        plsc.subcore_barrier()

        @pl.when(s < _NS - 1)
        def _():
            pltpu.sync_copy(acc.at[pl.ds(r0, _RPTA)],
                            out_hbm.at[p, c, pl.ds(r0, _RPTA)])

        @pl.when(s == _NS - 1)
        def _():
            pltpu.sync_copy(acc.at[pl.ds(r0, _RPTL)],
                            out_hbm.at[p, c, pl.ds(r0, _RPTL)])

        plsc.subcore_barrier()


_sc_edge = functools.partial(
    pl.kernel,
    out_type=jax.ShapeDtypeStruct((2, _NC, _HR, HID), jnp.float32),
    mesh=plsc.VectorSubcoreMesh(core_axis_name="c", subcore_axis_name="s"),
    compiler_params=pltpu.CompilerParams(use_tc_tiling_on_sc=False),
    scratch_types=[
        pltpu.VMEM((_KB, _IB), jnp.int32),
        pltpu.VMEM((_KB, _IB), jnp.int32),
        pltpu.VMEM((_KB, _IB), jnp.int32),
        pltpu.VMEM((_EPC, HID), jnp.float32),
        pltpu.VMEM((_EPC, HID), jnp.float32),
        pltpu.VMEM_SHARED((_HR + 128, HID), jnp.float32),
        pltpu.SemaphoreType.DMA,
    ],
)(_sc_body)


_PB = _HR // _BLK


def _decode_body(outp_ref, acc_ref, wnd2_ref, out_ref):
    agg = acc_ref[0, 0] + acc_ref[0, 1]
    vel = jnp.dot(agg, wnd2_ref[...], preferred_element_type=jnp.float32)
    out_ref[...] = outp_ref[...] + vel


_decode = pl.pallas_call(
    _decode_body,
    grid=(_NBLK,),
    in_specs=[
        pl.BlockSpec((_BLK, MD), lambda i: (i, 0)),
        pl.BlockSpec((1, _NC, _BLK, HID), lambda i: (i // _PB, 0, i % _PB, 0)),
        pl.BlockSpec((HID, MD), lambda i: (0, 0)),
    ],
    out_specs=pl.BlockSpec((_BLK, MD), lambda i: (i, 0)),
    out_shape=jax.ShapeDtypeStruct((N, MD), jnp.float32),
)


def kernel(t, node_f, edge_index, xf, W_ode, b_ode, W_ne, b_ne,
           W_ee, b_ee, W_nd, b_nd):
    del t
    nf = jnp.concatenate([xf, node_f], axis=1)
    wnd_pad = jnp.pad(W_nd, ((0, 0), (MD - 2, 0)))
    bode_eff = (b_ode.reshape(1, MD)
                + jnp.pad(b_nd, (MD - 2, 0)).reshape(1, MD))
    A, Bb, outp = _encode(
        nf, W_ne, b_ne.reshape(1, HID), W_ee, b_ee.reshape(1, HID),
        W_ode, bode_eff, wnd_pad[:HID])
    src2 = edge_index[0].reshape(E // _IB, _IB)
    dst2 = edge_index[1].reshape(E // _IB, _IB)
    acc4 = _sc_edge(src2, dst2, A, Bb)
    return _decode(outp, acc4, wnd_pad[HID:])


# speedup vs baseline: 15.9117x; 1.9782x over previous
import functools

import jax
import jax.numpy as jnp
from jax import lax
from jax.experimental import pallas as pl
from jax.experimental.pallas import tpu as pltpu
from jax.experimental.pallas import tpu_sc as plsc

N = 100000
E = 1600000
MD = 4
HID = 16


_BLK = 2000
_NBLK = N // _BLK


def _encode_body(nf_ref, wne_ref, bne_ref, wee_ref, bee_ref, wode_ref,
                 bode_ref, wnd1_ref, a_ref, b_ref, outp_ref):
    nf = nf_ref[...]
    ne = jnp.dot(nf, wne_ref[...], preferred_element_type=jnp.float32)
    ne = jnp.maximum(ne + bne_ref[...], 0.0)
    wee = wee_ref[...]
    a_ref[...] = jnp.dot(ne, wee[:HID], preferred_element_type=jnp.float32)
    b_ref[...] = (jnp.dot(ne, wee[HID:], preferred_element_type=jnp.float32)
                  + bee_ref[...])
    node_f = nf[:, 2:2 + MD]
    base = jnp.dot(node_f, wode_ref[...], preferred_element_type=jnp.float32)
    vel = jnp.dot(ne, wnd1_ref[...], preferred_element_type=jnp.float32)
    outp_ref[...] = base + bode_ref[...] + vel


_encode = pl.pallas_call(
    _encode_body,
    grid=(_NBLK,),
    in_specs=[
        pl.BlockSpec((_BLK, 2 + MD), lambda i: (i, 0)),
        pl.BlockSpec((2 + MD, HID), lambda i: (0, 0)),
        pl.BlockSpec((1, HID), lambda i: (0, 0)),
        pl.BlockSpec((2 * HID, HID), lambda i: (0, 0)),
        pl.BlockSpec((1, HID), lambda i: (0, 0)),
        pl.BlockSpec((MD, MD), lambda i: (0, 0)),
        pl.BlockSpec((1, MD), lambda i: (0, 0)),
        pl.BlockSpec((HID, MD), lambda i: (0, 0)),
    ],
    out_specs=[
        pl.BlockSpec((_BLK, HID), lambda i: (i, 0)),
        pl.BlockSpec((_BLK, HID), lambda i: (i, 0)),
        pl.BlockSpec((_BLK, MD), lambda i: (i, 0)),
    ],
    out_shape=[
        jax.ShapeDtypeStruct((N, HID), jnp.float32),
        jax.ShapeDtypeStruct((N, HID), jnp.float32),
        jax.ShapeDtypeStruct((N, MD), jnp.float32),
    ],
)


_NC = 2
_NS = 16
_NW = _NC * _NS
_IB = 128
_KB = 8
_EPC = _IB * _KB
_NROW = E // _IB
_NCH = _NROW // _KB
_TKB = _NROW - _NCH * _KB
_CH_PER_W = (_NCH + _NW - 1) // _NW
_HR = N // 2
_RPTA = 3136
_RPTL = _HR - (_NS - 1) * _RPTA


_NPAIR = (_CH_PER_W + 1) // 2


def _sc_body(src_hbm, dst_hbm, a_hbm, b_hbm, out_hbm,
             si0, di0, d2i0, si1, di1, d2i1,
             a0, b0, a1, b1, acc,
             sem_g0, sem_g1, sem_s0, sem_s1, sem_i0, sem_i1):
    c = lax.axis_index("c")
    s = lax.axis_index("s")
    wid = c * _NS + s
    r0 = s * _RPTA

    def _valid(t):
        return (t * _NW + wid) < _NCH

    def _row0(t):
        return (t * _NW + wid) * _KB

    def _fire_idx(t, si, di, sem):
        pltpu.async_copy(src_hbm.at[pl.ds(_row0(t), _KB)], si.at[...], sem)
        pltpu.async_copy(dst_hbm.at[pl.ds(_row0(t), _KB)], di.at[...], sem)

    def _wait_idx(t, si, di, sem):
        pltpu.make_async_copy(
            src_hbm.at[pl.ds(_row0(t), _KB)], si.at[...], sem).wait()
        pltpu.make_async_copy(
            dst_hbm.at[pl.ds(_row0(t), _KB)], di.at[...], sem).wait()

    def _fire_gathers(si, di, a_r, b_r, sem):
        for j in range(_KB):
            pltpu.async_copy(a_hbm.at[si.at[j]],
                             a_r.at[pl.ds(j * _IB, _IB)], sem)
            pltpu.async_copy(b_hbm.at[di.at[j]],
                             b_r.at[pl.ds(j * _IB, _IB)], sem)

    def _wait_gathers(si, di, a_r, b_r, sem):
        for j in range(_KB):
            pltpu.make_async_copy(a_hbm.at[si.at[j]],
                                  a_r.at[pl.ds(j * _IB, _IB)], sem).wait()
            pltpu.make_async_copy(b_hbm.at[di.at[j]],
                                  b_r.at[pl.ds(j * _IB, _IB)], sem).wait()

    def _fire_scatters(v_r, d2i, sem):
        for j in range(_KB):
            pltpu.async_copy(v_r.at[pl.ds(j * _IB, _IB)],
                             acc.at[d2i.at[j]], sem, add=True)

    def _wait_scatters(v_r, d2i, sem):
        for j in range(_KB):
            pltpu.make_async_copy(v_r.at[pl.ds(j * _IB, _IB)],
                                  acc.at[d2i.at[j]], sem).wait()

    def _process(di, d2i, a_r, b_r, lo, kb=_KB):
        for j in range(kb):
            for k in range(_IB // 16):
                d = di[j, pl.ds(k * 16, 16)] - lo
                dummy = _HR + (k % 8) * 16 + lax.iota(jnp.int32, 16)
                inr = (d >= 0) & (d < _HR)
                d2i[j, pl.ds(k * 16, 16)] = jnp.where(inr, d, dummy)

        @plsc.parallel_loop(0, kb * _IB, unroll=8)
        def _edge(e):
            a_r[e] = jnp.maximum(a_r[e] + b_r[e], 0.0)

    def _zero_fill(i, carry):
        a0[i] = jnp.zeros((HID,), jnp.float32)
        return carry

    def _zero_slice(cnt):
        nfull = cnt // _EPC
        for k in range(nfull):
            pltpu.sync_copy(a0.at[...],
                            acc.at[pl.ds(r0 + k * _EPC, _EPC)])
        rem = cnt - nfull * _EPC
        if rem:
            pltpu.sync_copy(a0.at[pl.ds(0, rem)],
                            acc.at[pl.ds(r0 + nfull * _EPC, rem)])

    for p in range(2):
        lo = p * _HR
        lax.fori_loop(0, _EPC, _zero_fill, 0)

        @pl.when(s < _NS - 1)
        def _():
            _zero_slice(_RPTA)

        @pl.when(s == _NS - 1)
        def _():
            _zero_slice(_RPTL)

        plsc.subcore_barrier()

        @pl.when(_valid(0))
        def _():
            _fire_idx(0, si0, di0, sem_i0)
            _wait_idx(0, si0, di0, sem_i0)
            _fire_gathers(si0, di0, a0, b0, sem_g0)

        @pl.when(_valid(1))
        def _():
            _fire_idx(1, si1, di1, sem_i1)

        def _pair(u, carry, lo=lo):
            t0 = 2 * u
            t1 = 2 * u + 1

            @pl.when(_valid(t0))
            def _():
                _wait_gathers(si0, di0, a0, b0, sem_g0)

            @pl.when((u > 0) & _valid(t1 - 2))
            def _():
                _wait_scatters(a1, d2i1, sem_s1)

            @pl.when(_valid(t1))
            def _():
                _wait_idx(t1, si1, di1, sem_i1)
                _fire_gathers(si1, di1, a1, b1, sem_g1)

            @pl.when(_valid(t0))
            def _():
                _process(di0, d2i0, a0, b0, lo)
                _fire_scatters(a0, d2i0, sem_s0)

            @pl.when(_valid(t0 + 2))
            def _():
                _fire_idx(t0 + 2, si0, di0, sem_i0)

            @pl.when(_valid(t1))
            def _():
                _wait_gathers(si1, di1, a1, b1, sem_g1)

            @pl.when(_valid(t0 + 2))
            def _():
                _wait_scatters(a0, d2i0, sem_s0)
                _wait_idx(t0 + 2, si0, di0, sem_i0)
                _fire_gathers(si0, di0, a0, b0, sem_g0)

            @pl.when(_valid(t1))
            def _():
                _process(di1, d2i1, a1, b1, lo)
                _fire_scatters(a1, d2i1, sem_s1)

            @pl.when(_valid(t1 + 2))
            def _():
                _fire_idx(t1 + 2, si1, di1, sem_i1)

            return carry

        lax.fori_loop(0, _NPAIR, _pair, 0)

        _wait_scatters(a0, d2i0, sem_s0)

        if _TKB:
            @pl.when(wid == 0)
            def _tail():
                row0 = _NCH * _KB
                pltpu.sync_copy(src_hbm.at[pl.ds(row0, _TKB)],
                                si0.at[pl.ds(0, _TKB)])
                pltpu.sync_copy(dst_hbm.at[pl.ds(row0, _TKB)],
                                di0.at[pl.ds(0, _TKB)])
                cps = []
                for j in range(_TKB):
                    cps.append(pltpu.async_copy(
                        a_hbm.at[si0.at[j]],
                        a0.at[pl.ds(j * _IB, _IB)], sem_g0))
                    cps.append(pltpu.async_copy(
                        b_hbm.at[di0.at[j]],
                        b0.at[pl.ds(j * _IB, _IB)], sem_g0))
                for cp in cps:
                    cp.wait()
                _process(di0, d2i0, a0, b0, lo, kb=_TKB)
                scs = []
                for j in range(_TKB):
                    scs.append(pltpu.async_copy(
                        a0.at[pl.ds(j * _IB, _IB)],
                        acc.at[d2i0.at[j]], sem_s0, add=True))
                for cp in scs:
                    cp.wait()

        plsc.subcore_barrier()

        @pl.when(s < _NS - 1)
        def _():
            pltpu.sync_copy(acc.at[pl.ds(r0, _RPTA)],
                            out_hbm.at[p, c, pl.ds(r0, _RPTA)])

        @pl.when(s == _NS - 1)
        def _():
            pltpu.sync_copy(acc.at[pl.ds(r0, _RPTL)],
                            out_hbm.at[p, c, pl.ds(r0, _RPTL)])

        plsc.subcore_barrier()


_sc_edge = functools.partial(
    pl.kernel,
    out_type=jax.ShapeDtypeStruct((2, _NC, _HR, HID), jnp.float32),
    mesh=plsc.VectorSubcoreMesh(core_axis_name="c", subcore_axis_name="s"),
    compiler_params=pltpu.CompilerParams(use_tc_tiling_on_sc=False),
    scratch_types=[
        pltpu.VMEM((_KB, _IB), jnp.int32),
        pltpu.VMEM((_KB, _IB), jnp.int32),
        pltpu.VMEM((_KB, _IB), jnp.int32),
        pltpu.VMEM((_KB, _IB), jnp.int32),
        pltpu.VMEM((_KB, _IB), jnp.int32),
        pltpu.VMEM((_KB, _IB), jnp.int32),
        pltpu.VMEM((_EPC, HID), jnp.float32),
        pltpu.VMEM((_EPC, HID), jnp.float32),
        pltpu.VMEM((_EPC, HID), jnp.float32),
        pltpu.VMEM((_EPC, HID), jnp.float32),
        pltpu.VMEM_SHARED((_HR + 128, HID), jnp.float32),
        pltpu.SemaphoreType.DMA,
        pltpu.SemaphoreType.DMA,
        pltpu.SemaphoreType.DMA,
        pltpu.SemaphoreType.DMA,
        pltpu.SemaphoreType.DMA,
        pltpu.SemaphoreType.DMA,
    ],
)(_sc_body)


_PB = _HR // _BLK


def _decode_body(outp_ref, acc_ref, wnd2_ref, out_ref):
    agg = acc_ref[0, 0] + acc_ref[0, 1]
    vel = jnp.dot(agg, wnd2_ref[...], preferred_element_type=jnp.float32)
    out_ref[...] = outp_ref[...] + vel


_decode = pl.pallas_call(
    _decode_body,
    grid=(_NBLK,),
    in_specs=[
        pl.BlockSpec((_BLK, MD), lambda i: (i, 0)),
        pl.BlockSpec((1, _NC, _BLK, HID), lambda i: (i // _PB, 0, i % _PB, 0)),
        pl.BlockSpec((HID, MD), lambda i: (0, 0)),
    ],
    out_specs=pl.BlockSpec((_BLK, MD), lambda i: (i, 0)),
    out_shape=jax.ShapeDtypeStruct((N, MD), jnp.float32),
)


def kernel(t, node_f, edge_index, xf, W_ode, b_ode, W_ne, b_ne,
           W_ee, b_ee, W_nd, b_nd):
    del t
    nf = jnp.concatenate([xf, node_f], axis=1)
    wnd_pad = jnp.pad(W_nd, ((0, 0), (MD - 2, 0)))
    bode_eff = (b_ode.reshape(1, MD)
                + jnp.pad(b_nd, (MD - 2, 0)).reshape(1, MD))
    A, Bb, outp = _encode(
        nf, W_ne, b_ne.reshape(1, HID), W_ee, b_ee.reshape(1, HID),
        W_ode, bode_eff, wnd_pad[:HID])
    src2 = edge_index[0].reshape(E // _IB, _IB)
    dst2 = edge_index[1].reshape(E // _IB, _IB)
    acc4 = _sc_edge(src2, dst2, A, Bb)
    return _decode(outp, acc4, wnd_pad[HID:])


# speedup vs baseline: 20.2184x; 1.2707x over previous
import functools

import jax
import jax.numpy as jnp
from jax import lax
from jax.experimental import pallas as pl
from jax.experimental.pallas import tpu as pltpu
from jax.experimental.pallas import tpu_sc as plsc

N = 100000
E = 1600000
MD = 4
HID = 16


_BLK = 2000
_NBLK = N // _BLK


def _encode_body(nfp_ref, wne_ref, bne_ref, wtop_ref, wbot_ref, bee_ref,
                 wode_ref, bode_ref, wnd1_ref, a_ref, b_ref, outp_ref):
    nfp = nfp_ref[0]
    ne = jnp.dot(nfp, wne_ref[...], preferred_element_type=jnp.float32)
    ne = jnp.maximum(ne + bne_ref[...], 0.0)
    a_ref[...] = jnp.dot(
        ne, wtop_ref[...], preferred_element_type=jnp.float32
    ).reshape(1, _RB, 128)
    b_ref[...] = (jnp.dot(ne, wbot_ref[...],
                          preferred_element_type=jnp.float32)
                  + bee_ref[...]).reshape(1, _RB, 128)
    base = jnp.dot(nfp, wode_ref[...], preferred_element_type=jnp.float32)
    vel = jnp.dot(ne, wnd1_ref[...], preferred_element_type=jnp.float32)
    outp_ref[...] = (base + bode_ref[...] + vel).reshape(1, _RB, 32)


_RB = _BLK // 8

_encode = pl.pallas_call(
    _encode_body,
    grid=(_NBLK,),
    in_specs=[
        pl.BlockSpec((1, _RB, 48), lambda i: (i, 0, 0)),
        pl.BlockSpec((48, 128), lambda i: (0, 0)),
        pl.BlockSpec((1, 128), lambda i: (0, 0)),
        pl.BlockSpec((128, 128), lambda i: (0, 0)),
        pl.BlockSpec((128, 128), lambda i: (0, 0)),
        pl.BlockSpec((1, 128), lambda i: (0, 0)),
        pl.BlockSpec((48, 32), lambda i: (0, 0)),
        pl.BlockSpec((1, 32), lambda i: (0, 0)),
        pl.BlockSpec((128, 32), lambda i: (0, 0)),
    ],
    out_specs=[
        pl.BlockSpec((1, _RB, 128), lambda i: (i, 0, 0)),
        pl.BlockSpec((1, _RB, 128), lambda i: (i, 0, 0)),
        pl.BlockSpec((1, _RB, 32), lambda i: (i, 0, 0)),
    ],
    out_shape=[
        jax.ShapeDtypeStruct((_NBLK, _RB, 128), jnp.float32),
        jax.ShapeDtypeStruct((_NBLK, _RB, 128), jnp.float32),
        jax.ShapeDtypeStruct((_NBLK, _RB, 32), jnp.float32),
    ],
)


_NC = 2
_NS = 16
_NW = _NC * _NS
_IB = 128
_KB = 8
_EPC = _IB * _KB
_NROW = E // _IB
_NCH = _NROW // _KB
_TKB = _NROW - _NCH * _KB
_CH_PER_W = (_NCH + _NW - 1) // _NW
_HR = N // 2
_RPTA = 3136
_RPTL = _HR - (_NS - 1) * _RPTA


_NPAIR = (_CH_PER_W + 1) // 2


def _sc_body(ei_hbm, a_hbm, b_hbm, out_hbm,
             si0, di0, d2i0, si1, di1, d2i1,
             a0, b0, a1, b1, acc,
             sem_g0, sem_g1, sem_s0, sem_s1, sem_i0, sem_i1):
    c = lax.axis_index("c")
    s = lax.axis_index("s")
    wid = c * _NS + s
    r0 = s * _RPTA

    def _valid(t):
        return (t * _NW + wid) < _NCH

    def _e0(t):
        return (t * _NW + wid) * _EPC

    def _fire_idx(t, si, di, sem):
        pltpu.async_copy(ei_hbm.at[0, pl.ds(_e0(t), _EPC)], si.at[...], sem)
        pltpu.async_copy(ei_hbm.at[1, pl.ds(_e0(t), _EPC)], di.at[...], sem)

    def _wait_idx(t, si, di, sem):
        pltpu.make_async_copy(
            ei_hbm.at[0, pl.ds(_e0(t), _EPC)], si.at[...], sem).wait()
        pltpu.make_async_copy(
            ei_hbm.at[1, pl.ds(_e0(t), _EPC)], di.at[...], sem).wait()

    def _fire_gathers(si, di, a_r, b_r, sem):
        for j in range(_KB):
            pltpu.async_copy(a_hbm.at[si.at[pl.ds(j * _IB, _IB)]],
                             a_r.at[pl.ds(j * _IB, _IB)], sem)
            pltpu.async_copy(b_hbm.at[di.at[pl.ds(j * _IB, _IB)]],
                             b_r.at[pl.ds(j * _IB, _IB)], sem)

    def _wait_gathers(si, di, a_r, b_r, sem):
        for j in range(_KB):
            pltpu.make_async_copy(a_hbm.at[si.at[pl.ds(j * _IB, _IB)]],
                                  a_r.at[pl.ds(j * _IB, _IB)], sem).wait()
            pltpu.make_async_copy(b_hbm.at[di.at[pl.ds(j * _IB, _IB)]],
                                  b_r.at[pl.ds(j * _IB, _IB)], sem).wait()

    def _fire_scatters(v_r, d2i, sem):
        for j in range(_KB):
            pltpu.async_copy(v_r.at[pl.ds(j * _IB, _IB)],
                             acc.at[d2i.at[j]], sem, add=True)

    def _wait_scatters(v_r, d2i, sem):
        for j in range(_KB):
            pltpu.make_async_copy(v_r.at[pl.ds(j * _IB, _IB)],
                                  acc.at[d2i.at[j]], sem).wait()

    def _process(di, d2i, a_r, b_r, lo, kb=_KB):
        for j in range(kb):
            for k in range(_IB // 16):
                d = di[pl.ds(j * _IB + k * 16, 16)] - lo
                dummy = _HR + (k % 8) * 16 + lax.iota(jnp.int32, 16)
                inr = (d >= 0) & (d < _HR)
                d2i[j, pl.ds(k * 16, 16)] = jnp.where(inr, d, dummy)

        @plsc.parallel_loop(0, kb * _IB, unroll=8)
        def _edge(e):
            a_r[e] = jnp.maximum(a_r[e] + b_r[e], 0.0)

    def _zero_fill(i, carry):
        a0[i] = jnp.zeros((HID,), jnp.float32)
        return carry

    def _zero_slice(cnt):
        nfull = cnt // _EPC
        for k in range(nfull):
            pltpu.sync_copy(a0.at[...],
                            acc.at[pl.ds(r0 + k * _EPC, _EPC)])
        rem = cnt - nfull * _EPC
        if rem:
            pltpu.sync_copy(a0.at[pl.ds(0, rem)],
                            acc.at[pl.ds(r0 + nfull * _EPC, rem)])

    for p in range(2):
        lo = p * _HR
        lax.fori_loop(0, _EPC, _zero_fill, 0)

        @pl.when(s < _NS - 1)
        def _():
            _zero_slice(_RPTA)

        @pl.when(s == _NS - 1)
        def _():
            _zero_slice(_RPTL)

        plsc.subcore_barrier()

        @pl.when(_valid(0))
        def _():
            _fire_idx(0, si0, di0, sem_i0)
            _wait_idx(0, si0, di0, sem_i0)
            _fire_gathers(si0, di0, a0, b0, sem_g0)

        @pl.when(_valid(1))
        def _():
            _fire_idx(1, si1, di1, sem_i1)

        def _pair(u, carry, lo=lo):
            t0 = 2 * u
            t1 = 2 * u + 1

            @pl.when(_valid(t0))
            def _():
                _wait_gathers(si0, di0, a0, b0, sem_g0)

            @pl.when((u > 0) & _valid(t1 - 2))
            def _():
                _wait_scatters(a1, d2i1, sem_s1)

            @pl.when(_valid(t1))
            def _():
                _wait_idx(t1, si1, di1, sem_i1)
                _fire_gathers(si1, di1, a1, b1, sem_g1)

            @pl.when(_valid(t0))
            def _():
                _process(di0, d2i0, a0, b0, lo)
                _fire_scatters(a0, d2i0, sem_s0)

            @pl.when(_valid(t0 + 2))
            def _():
                _fire_idx(t0 + 2, si0, di0, sem_i0)

            @pl.when(_valid(t1))
            def _():
                _wait_gathers(si1, di1, a1, b1, sem_g1)

            @pl.when(_valid(t0 + 2))
            def _():
                _wait_scatters(a0, d2i0, sem_s0)
                _wait_idx(t0 + 2, si0, di0, sem_i0)
                _fire_gathers(si0, di0, a0, b0, sem_g0)

            @pl.when(_valid(t1))
            def _():
                _process(di1, d2i1, a1, b1, lo)
                _fire_scatters(a1, d2i1, sem_s1)

            @pl.when(_valid(t1 + 2))
            def _():
                _fire_idx(t1 + 2, si1, di1, sem_i1)

            return carry

        lax.fori_loop(0, _NPAIR, _pair, 0)

        _wait_scatters(a0, d2i0, sem_s0)

        if _TKB:
            @pl.when(wid == 0)
            def _tail():
                te0 = _NCH * _EPC
                tne = _TKB * _IB
                pltpu.sync_copy(ei_hbm.at[0, pl.ds(te0, tne)],
                                si0.at[pl.ds(0, tne)])
                pltpu.sync_copy(ei_hbm.at[1, pl.ds(te0, tne)],
                                di0.at[pl.ds(0, tne)])
                cps = []
                for j in range(_TKB):
                    cps.append(pltpu.async_copy(
                        a_hbm.at[si0.at[pl.ds(j * _IB, _IB)]],
                        a0.at[pl.ds(j * _IB, _IB)], sem_g0))
                    cps.append(pltpu.async_copy(
                        b_hbm.at[di0.at[pl.ds(j * _IB, _IB)]],
                        b0.at[pl.ds(j * _IB, _IB)], sem_g0))
                for cp in cps:
                    cp.wait()
                _process(di0, d2i0, a0, b0, lo, kb=_TKB)
                scs = []
                for j in range(_TKB):
                    scs.append(pltpu.async_copy(
                        a0.at[pl.ds(j * _IB, _IB)],
                        acc.at[d2i0.at[j]], sem_s0, add=True))
                for cp in scs:
                    cp.wait()

        plsc.subcore_barrier()

        @pl.when(s < _NS - 1)
        def _():
            pltpu.sync_copy(acc.at[pl.ds(r0, _RPTA)],
                            out_hbm.at[p, c, pl.ds(r0, _RPTA)])

        @pl.when(s == _NS - 1)
        def _():
            pltpu.sync_copy(acc.at[pl.ds(r0, _RPTL)],
                            out_hbm.at[p, c, pl.ds(r0, _RPTL)])

        plsc.subcore_barrier()


_sc_edge = functools.partial(
    pl.kernel,
    out_type=jax.ShapeDtypeStruct((2, _NC, _HR, HID), jnp.float32),
    mesh=plsc.VectorSubcoreMesh(core_axis_name="c", subcore_axis_name="s"),
    compiler_params=pltpu.CompilerParams(use_tc_tiling_on_sc=False),
    scratch_types=[
        pltpu.VMEM((_EPC,), jnp.int32),
        pltpu.VMEM((_EPC,), jnp.int32),
        pltpu.VMEM((_KB, _IB), jnp.int32),
        pltpu.VMEM((_EPC,), jnp.int32),
        pltpu.VMEM((_EPC,), jnp.int32),
        pltpu.VMEM((_KB, _IB), jnp.int32),
        pltpu.VMEM((_EPC, HID), jnp.float32),
        pltpu.VMEM((_EPC, HID), jnp.float32),
        pltpu.VMEM((_EPC, HID), jnp.float32),
        pltpu.VMEM((_EPC, HID), jnp.float32),
        pltpu.VMEM_SHARED((_HR + 128, HID), jnp.float32),
        pltpu.SemaphoreType.DMA,
        pltpu.SemaphoreType.DMA,
        pltpu.SemaphoreType.DMA,
        pltpu.SemaphoreType.DMA,
        pltpu.SemaphoreType.DMA,
        pltpu.SemaphoreType.DMA,
    ],
)(_sc_body)


_PB = _HR // _BLK


def _decode_body(outp_ref, acc_ref, wnd2_ref, out_ref):
    agg = acc_ref[0, 0, 0] + acc_ref[0, 1, 0]
    vel = jnp.dot(agg, wnd2_ref[...], preferred_element_type=jnp.float32)
    out_ref[...] = outp_ref[...] + vel.reshape(1, _RB, 32)


_decode = pl.pallas_call(
    _decode_body,
    grid=(_NBLK,),
    in_specs=[
        pl.BlockSpec((1, _RB, 32), lambda i: (i, 0, 0)),
        pl.BlockSpec((1, _NC, 1, _RB, 128),
                     lambda i: (i // _PB, 0, i % _PB, 0, 0)),
        pl.BlockSpec((128, 32), lambda i: (0, 0)),
    ],
    out_specs=pl.BlockSpec((1, _RB, 32), lambda i: (i, 0, 0)),
    out_shape=jax.ShapeDtypeStruct((_NBLK, _RB, 32), jnp.float32),
)


def kernel(t, node_f, edge_index, xf, W_ode, b_ode, W_ne, b_ne,
           W_ee, b_ee, W_nd, b_nd):
    del t
    wnd_pad = jnp.pad(W_nd, ((0, 0), (MD - 2, 0)))
    bode_eff = b_ode + jnp.pad(b_nd, (MD - 2, 0))
    eye8 = jnp.eye(8, dtype=jnp.float32)
    wne_bd = jnp.kron(eye8, W_ne)
    wtop_bd = jnp.kron(eye8, W_ee[:HID])
    wbot_bd = jnp.kron(eye8, W_ee[HID:])
    wnd1_bd = jnp.kron(eye8, wnd_pad[:HID])
    wnd2_bd = jnp.kron(eye8, wnd_pad[HID:])
    wode_bd = jnp.kron(
        eye8, jnp.concatenate([jnp.zeros((2, MD), jnp.float32), W_ode]))
    bne_t = jnp.tile(b_ne, 8).reshape(1, 128)
    bee_t = jnp.tile(b_ee, 8).reshape(1, 128)
    bode_t = jnp.tile(bode_eff, 8).reshape(1, 32)
    nfp = jnp.concatenate([xf, node_f], axis=1).reshape(_NBLK, _RB, 48)
    A2, B2, outp = _encode(nfp, wne_bd, bne_t, wtop_bd, wbot_bd, bee_t,
                           wode_bd, bode_t, wnd1_bd)
    acc4 = _sc_edge(edge_index, A2.reshape(N, HID), B2.reshape(N, HID))
    acc4r = acc4.reshape(2, _NC, _PB, _RB, 128)
    outf = _decode(outp, acc4r, wnd2_bd)
    return outf.reshape(N, MD)


# speedup vs baseline: 20.2850x; 1.0033x over previous
import functools

import jax
import jax.numpy as jnp
from jax import lax
from jax.experimental import pallas as pl
from jax.experimental.pallas import tpu as pltpu
from jax.experimental.pallas import tpu_sc as plsc

N = 100000
E = 1600000
MD = 4
HID = 16


_BLK = 2000
_NBLK = N // _BLK


def _encode_body(nfp_ref, wne_ref, bne_ref, wtop_ref, wbot_ref, bee_ref,
                 wode_ref, bode_ref, wnd1_ref, a_ref, b_ref, outp_ref):
    nfp = nfp_ref[0]
    ne = jnp.dot(nfp, wne_ref[...], preferred_element_type=jnp.float32)
    ne = jnp.maximum(ne + bne_ref[...], 0.0)
    a_ref[...] = jnp.dot(
        ne, wtop_ref[...], preferred_element_type=jnp.float32
    ).reshape(1, _RB, 128)
    b_ref[...] = (jnp.dot(ne, wbot_ref[...],
                          preferred_element_type=jnp.float32)
                  + bee_ref[...]).reshape(1, _RB, 128)
    base = jnp.dot(nfp, wode_ref[...], preferred_element_type=jnp.float32)
    vel = jnp.dot(ne, wnd1_ref[...], preferred_element_type=jnp.float32)
    outp_ref[...] = (base + bode_ref[...] + vel).reshape(1, _RB, 32)


_RB = _BLK // 8

_encode = pl.pallas_call(
    _encode_body,
    grid=(_NBLK,),
    in_specs=[
        pl.BlockSpec((1, _RB, 48), lambda i: (i, 0, 0)),
        pl.BlockSpec((48, 128), lambda i: (0, 0)),
        pl.BlockSpec((1, 128), lambda i: (0, 0)),
        pl.BlockSpec((128, 128), lambda i: (0, 0)),
        pl.BlockSpec((128, 128), lambda i: (0, 0)),
        pl.BlockSpec((1, 128), lambda i: (0, 0)),
        pl.BlockSpec((48, 32), lambda i: (0, 0)),
        pl.BlockSpec((1, 32), lambda i: (0, 0)),
        pl.BlockSpec((128, 32), lambda i: (0, 0)),
    ],
    out_specs=[
        pl.BlockSpec((1, _RB, 128), lambda i: (i, 0, 0)),
        pl.BlockSpec((1, _RB, 128), lambda i: (i, 0, 0)),
        pl.BlockSpec((1, _RB, 32), lambda i: (i, 0, 0)),
    ],
    out_shape=[
        jax.ShapeDtypeStruct((_NBLK, _RB, 128), jnp.float32),
        jax.ShapeDtypeStruct((_NBLK, _RB, 128), jnp.float32),
        jax.ShapeDtypeStruct((_NBLK, _RB, 32), jnp.float32),
    ],
)


_NC = 2
_NS = 16
_NW = _NC * _NS
_IB = 128
_KB = 8
_EPC = _IB * _KB
_NROW = E // _IB
_NCH = _NROW // _KB
_TKB = _NROW - _NCH * _KB
_CH_PER_W = (_NCH + _NW - 1) // _NW
_HR = N // 2
_RPTA = 3136
_RPTL = _HR - (_NS - 1) * _RPTA


_NPAIR = (_CH_PER_W + 1) // 2


def _sc_body(ei_hbm, a_hbm, b_hbm, out_hbm,
             si0, di0, d2i0, si1, di1, d2i1,
             a0, b0, a1, b1, acc,
             sem_g0, sem_g1, sem_s0, sem_s1, sem_i0, sem_i1):
    c = lax.axis_index("c")
    s = lax.axis_index("s")
    wid = c * _NS + s
    r0 = s * _RPTA

    def _valid(t):
        return (t * _NW + wid) < _NCH

    def _e0(t):
        return (t * _NW + wid) * _EPC

    def _fire_idx(t, si, di, sem):
        pltpu.async_copy(ei_hbm.at[0, pl.ds(_e0(t), _EPC)], si.at[...], sem)
        pltpu.async_copy(ei_hbm.at[1, pl.ds(_e0(t), _EPC)], di.at[...], sem)

    def _wait_idx(t, si, di, sem):
        pltpu.make_async_copy(
            ei_hbm.at[0, pl.ds(_e0(t), _EPC)], si.at[...], sem).wait()
        pltpu.make_async_copy(
            ei_hbm.at[1, pl.ds(_e0(t), _EPC)], di.at[...], sem).wait()

    def _fire_gathers(si, di, a_r, b_r, sem):
        pltpu.async_copy(a_hbm.at[si.at[...]], a_r.at[...], sem)
        pltpu.async_copy(b_hbm.at[di.at[...]], b_r.at[...], sem)

    def _wait_gathers(si, di, a_r, b_r, sem):
        pltpu.make_async_copy(a_hbm.at[si.at[...]], a_r.at[...], sem).wait()
        pltpu.make_async_copy(b_hbm.at[di.at[...]], b_r.at[...], sem).wait()

    def _fire_scatters(v_r, d2i, sem):
        for j in range(_KB):
            pltpu.async_copy(v_r.at[pl.ds(j * _IB, _IB)],
                             acc.at[d2i.at[j]], sem, add=True)

    def _wait_scatters(v_r, d2i, sem):
        for j in range(_KB):
            pltpu.make_async_copy(v_r.at[pl.ds(j * _IB, _IB)],
                                  acc.at[d2i.at[j]], sem).wait()

    def _process(di, d2i, a_r, b_r, lo, kb=_KB):
        for j in range(kb):
            for k in range(_IB // 16):
                d = di[pl.ds(j * _IB + k * 16, 16)] - lo
                dummy = _HR + (k % 8) * 16 + lax.iota(jnp.int32, 16)
                inr = (d >= 0) & (d < _HR)
                d2i[j, pl.ds(k * 16, 16)] = jnp.where(inr, d, dummy)

        @plsc.parallel_loop(0, kb * _IB, unroll=8)
        def _edge(e):
            a_r[e] = jnp.maximum(a_r[e] + b_r[e], 0.0)

    def _zero_fill(i, carry):
        a0[i] = jnp.zeros((HID,), jnp.float32)
        return carry

    def _zero_slice(cnt):
        nfull = cnt // _EPC
        for k in range(nfull):
            pltpu.sync_copy(a0.at[...],
                            acc.at[pl.ds(r0 + k * _EPC, _EPC)])
        rem = cnt - nfull * _EPC
        if rem:
            pltpu.sync_copy(a0.at[pl.ds(0, rem)],
                            acc.at[pl.ds(r0 + nfull * _EPC, rem)])

    for p in range(2):
        lo = p * _HR
        lax.fori_loop(0, _EPC, _zero_fill, 0)

        @pl.when(s < _NS - 1)
        def _():
            _zero_slice(_RPTA)

        @pl.when(s == _NS - 1)
        def _():
            _zero_slice(_RPTL)

        plsc.subcore_barrier()

        @pl.when(_valid(0))
        def _():
            _fire_idx(0, si0, di0, sem_i0)
            _wait_idx(0, si0, di0, sem_i0)
            _fire_gathers(si0, di0, a0, b0, sem_g0)

        @pl.when(_valid(1))
        def _():
            _fire_idx(1, si1, di1, sem_i1)

        def _pair(u, carry, lo=lo):
            t0 = 2 * u
            t1 = 2 * u + 1

            @pl.when(_valid(t0))
            def _():
                _wait_gathers(si0, di0, a0, b0, sem_g0)

            @pl.when((u > 0) & _valid(t1 - 2))
            def _():
                _wait_scatters(a1, d2i1, sem_s1)

            @pl.when(_valid(t1))
            def _():
                _wait_idx(t1, si1, di1, sem_i1)
                _fire_gathers(si1, di1, a1, b1, sem_g1)

            @pl.when(_valid(t0))
            def _():
                _process(di0, d2i0, a0, b0, lo)
                _fire_scatters(a0, d2i0, sem_s0)

            @pl.when(_valid(t0 + 2))
            def _():
                _fire_idx(t0 + 2, si0, di0, sem_i0)

            @pl.when(_valid(t1))
            def _():
                _wait_gathers(si1, di1, a1, b1, sem_g1)

            @pl.when(_valid(t0 + 2))
            def _():
                _wait_scatters(a0, d2i0, sem_s0)
                _wait_idx(t0 + 2, si0, di0, sem_i0)
                _fire_gathers(si0, di0, a0, b0, sem_g0)

            @pl.when(_valid(t1))
            def _():
                _process(di1, d2i1, a1, b1, lo)
                _fire_scatters(a1, d2i1, sem_s1)

            @pl.when(_valid(t1 + 2))
            def _():
                _fire_idx(t1 + 2, si1, di1, sem_i1)

            return carry

        lax.fori_loop(0, _NPAIR, _pair, 0)

        _wait_scatters(a0, d2i0, sem_s0)

        if _TKB:
            @pl.when(wid == 0)
            def _tail():
                te0 = _NCH * _EPC
                tne = _TKB * _IB
                pltpu.sync_copy(ei_hbm.at[0, pl.ds(te0, tne)],
                                si0.at[pl.ds(0, tne)])
                pltpu.sync_copy(ei_hbm.at[1, pl.ds(te0, tne)],
                                di0.at[pl.ds(0, tne)])
                cps = [pltpu.async_copy(
                           a_hbm.at[si0.at[pl.ds(0, tne)]],
                           a0.at[pl.ds(0, tne)], sem_g0),
                       pltpu.async_copy(
                           b_hbm.at[di0.at[pl.ds(0, tne)]],
                           b0.at[pl.ds(0, tne)], sem_g0)]
                for cp in cps:
                    cp.wait()
                _process(di0, d2i0, a0, b0, lo, kb=_TKB)
                scs = []
                for j in range(_TKB):
                    scs.append(pltpu.async_copy(
                        a0.at[pl.ds(j * _IB, _IB)],
                        acc.at[d2i0.at[j]], sem_s0, add=True))
                for cp in scs:
                    cp.wait()

        plsc.subcore_barrier()

        @pl.when(s < _NS - 1)
        def _():
            pltpu.sync_copy(acc.at[pl.ds(r0, _RPTA)],
                            out_hbm.at[p, c, pl.ds(r0, _RPTA)])

        @pl.when(s == _NS - 1)
        def _():
            pltpu.sync_copy(acc.at[pl.ds(r0, _RPTL)],
                            out_hbm.at[p, c, pl.ds(r0, _RPTL)])

        plsc.subcore_barrier()


_sc_edge = functools.partial(
    pl.kernel,
    out_type=jax.ShapeDtypeStruct((2, _NC, _HR, HID), jnp.float32),
    mesh=plsc.VectorSubcoreMesh(core_axis_name="c", subcore_axis_name="s"),
    compiler_params=pltpu.CompilerParams(use_tc_tiling_on_sc=False),
    scratch_types=[
        pltpu.VMEM((_EPC,), jnp.int32),
        pltpu.VMEM((_EPC,), jnp.int32),
        pltpu.VMEM((_KB, _IB), jnp.int32),
        pltpu.VMEM((_EPC,), jnp.int32),
        pltpu.VMEM((_EPC,), jnp.int32),
        pltpu.VMEM((_KB, _IB), jnp.int32),
        pltpu.VMEM((_EPC, HID), jnp.float32),
        pltpu.VMEM((_EPC, HID), jnp.float32),
        pltpu.VMEM((_EPC, HID), jnp.float32),
        pltpu.VMEM((_EPC, HID), jnp.float32),
        pltpu.VMEM_SHARED((_HR + 128, HID), jnp.float32),
        pltpu.SemaphoreType.DMA,
        pltpu.SemaphoreType.DMA,
        pltpu.SemaphoreType.DMA,
        pltpu.SemaphoreType.DMA,
        pltpu.SemaphoreType.DMA,
        pltpu.SemaphoreType.DMA,
    ],
)(_sc_body)


_PB = _HR // _BLK


def _decode_body(outp_ref, acc_ref, wnd2_ref, out_ref):
    agg = acc_ref[0, 0, 0] + acc_ref[0, 1, 0]
    vel = jnp.dot(agg, wnd2_ref[...], preferred_element_type=jnp.float32)
    out_ref[...] = outp_ref[...] + vel.reshape(1, _RB, 32)


_decode = pl.pallas_call(
    _decode_body,
    grid=(_NBLK,),
    in_specs=[
        pl.BlockSpec((1, _RB, 32), lambda i: (i, 0, 0)),
        pl.BlockSpec((1, _NC, 1, _RB, 128),
                     lambda i: (i // _PB, 0, i % _PB, 0, 0)),
        pl.BlockSpec((128, 32), lambda i: (0, 0)),
    ],
    out_specs=pl.BlockSpec((1, _RB, 32), lambda i: (i, 0, 0)),
    out_shape=jax.ShapeDtypeStruct((_NBLK, _RB, 32), jnp.float32),
)


def kernel(t, node_f, edge_index, xf, W_ode, b_ode, W_ne, b_ne,
           W_ee, b_ee, W_nd, b_nd):
    del t
    wnd_pad = jnp.pad(W_nd, ((0, 0), (MD - 2, 0)))
    bode_eff = b_ode + jnp.pad(b_nd, (MD - 2, 0))
    eye8 = jnp.eye(8, dtype=jnp.float32)
    wne_bd = jnp.kron(eye8, W_ne)
    wtop_bd = jnp.kron(eye8, W_ee[:HID])
    wbot_bd = jnp.kron(eye8, W_ee[HID:])
    wnd1_bd = jnp.kron(eye8, wnd_pad[:HID])
    wnd2_bd = jnp.kron(eye8, wnd_pad[HID:])
    wode_bd = jnp.kron(
        eye8, jnp.concatenate([jnp.zeros((2, MD), jnp.float32), W_ode]))
    bne_t = jnp.tile(b_ne, 8).reshape(1, 128)
    bee_t = jnp.tile(b_ee, 8).reshape(1, 128)
    bode_t = jnp.tile(bode_eff, 8).reshape(1, 32)
    nfp = jnp.concatenate([xf, node_f], axis=1).reshape(_NBLK, _RB, 48)
    A2, B2, outp = _encode(nfp, wne_bd, bne_t, wtop_bd, wbot_bd, bee_t,
                           wode_bd, bode_t, wnd1_bd)
    acc4 = _sc_edge(edge_index, A2.reshape(N, HID), B2.reshape(N, HID))
    acc4r = acc4.reshape(2, _NC, _PB, _RB, 128)
    outf = _decode(outp, acc4r, wnd2_bd)
    return outf.reshape(N, MD)


# speedup vs baseline: 21.6876x; 1.0691x over previous
import functools

import jax
import jax.numpy as jnp
from jax import lax
from jax.experimental import pallas as pl
from jax.experimental.pallas import tpu as pltpu
from jax.experimental.pallas import tpu_sc as plsc

N = 100000
E = 1600000
MD = 4
HID = 16


_BLK = 5000
_NBLK = N // _BLK


def _encode_body(nfp_ref, wne_ref, bne_ref, wtop_ref, wbot_ref, bee_ref,
                 wode_ref, bode_ref, wnd1_ref, a_ref, b_ref, outp_ref):
    nfp = nfp_ref[0]
    ne = jnp.dot(nfp, wne_ref[...], preferred_element_type=jnp.float32)
    ne = jnp.maximum(ne + bne_ref[...], 0.0)
    a_ref[...] = jnp.dot(
        ne, wtop_ref[...], preferred_element_type=jnp.float32
    ).reshape(1, _RB, 128)
    b_ref[...] = (jnp.dot(ne, wbot_ref[...],
                          preferred_element_type=jnp.float32)
                  + bee_ref[...]).reshape(1, _RB, 128)
    base = jnp.dot(nfp, wode_ref[...], preferred_element_type=jnp.float32)
    vel = jnp.dot(ne, wnd1_ref[...], preferred_element_type=jnp.float32)
    outp_ref[...] = (base + bode_ref[...] + vel).reshape(1, _RB, 32)


_RB = _BLK // 8

_encode = pl.pallas_call(
    _encode_body,
    grid=(_NBLK,),
    in_specs=[
        pl.BlockSpec((1, _RB, 48), lambda i: (i, 0, 0)),
        pl.BlockSpec((48, 128), lambda i: (0, 0)),
        pl.BlockSpec((1, 128), lambda i: (0, 0)),
        pl.BlockSpec((128, 128), lambda i: (0, 0)),
        pl.BlockSpec((128, 128), lambda i: (0, 0)),
        pl.BlockSpec((1, 128), lambda i: (0, 0)),
        pl.BlockSpec((48, 32), lambda i: (0, 0)),
        pl.BlockSpec((1, 32), lambda i: (0, 0)),
        pl.BlockSpec((128, 32), lambda i: (0, 0)),
    ],
    out_specs=[
        pl.BlockSpec((1, _RB, 128), lambda i: (i, 0, 0)),
        pl.BlockSpec((1, _RB, 128), lambda i: (i, 0, 0)),
        pl.BlockSpec((1, _RB, 32), lambda i: (i, 0, 0)),
    ],
    out_shape=[
        jax.ShapeDtypeStruct((_NBLK, _RB, 128), jnp.float32),
        jax.ShapeDtypeStruct((_NBLK, _RB, 128), jnp.float32),
        jax.ShapeDtypeStruct((_NBLK, _RB, 32), jnp.float32),
    ],
)


_NC = 2
_NS = 16
_NW = _NC * _NS
_IB = 128
_KB = 8
_EPC = _IB * _KB
_NROW = E // _IB
_NCH = _NROW // _KB
_TKB = _NROW - _NCH * _KB
_CH_PER_W = (_NCH + _NW - 1) // _NW
_HR = N // 2
_RPTA = 3136
_RPTL = _HR - (_NS - 1) * _RPTA


_NPAIR = (_CH_PER_W + 1) // 2


def _sc_body(ei_hbm, a_hbm, b_hbm, out_hbm,
             si0, di0, d2i0, si1, di1, d2i1,
             a0, b0, a1, b1, acc,
             sem_g0, sem_g1, sem_s0, sem_s1, sem_i0, sem_i1):
    c = lax.axis_index("c")
    s = lax.axis_index("s")
    wid = c * _NS + s
    r0 = s * _RPTA

    def _valid(t):
        return (t * _NW + wid) < _NCH

    def _e0(t):
        return (t * _NW + wid) * _EPC

    def _fire_idx(t, si, di, sem):
        pltpu.async_copy(ei_hbm.at[0, pl.ds(_e0(t), _EPC)], si.at[...], sem)
        pltpu.async_copy(ei_hbm.at[1, pl.ds(_e0(t), _EPC)], di.at[...], sem)

    def _wait_idx(t, si, di, sem):
        pltpu.make_async_copy(
            ei_hbm.at[0, pl.ds(_e0(t), _EPC)], si.at[...], sem).wait()
        pltpu.make_async_copy(
            ei_hbm.at[1, pl.ds(_e0(t), _EPC)], di.at[...], sem).wait()

    def _fire_gathers(si, di, a_r, b_r, sem):
        pltpu.async_copy(a_hbm.at[si.at[...]], a_r.at[...], sem)
        pltpu.async_copy(b_hbm.at[di.at[...]], b_r.at[...], sem)

    def _wait_gathers(si, di, a_r, b_r, sem):
        pltpu.make_async_copy(a_hbm.at[si.at[...]], a_r.at[...], sem).wait()
        pltpu.make_async_copy(b_hbm.at[di.at[...]], b_r.at[...], sem).wait()

    def _fire_scatters(v_r, d2i, sem):
        pltpu.async_copy(v_r.at[...], acc.at[d2i.at[...]], sem, add=True)

    def _wait_scatters(v_r, d2i, sem):
        pltpu.make_async_copy(v_r.at[...],
                              acc.at[d2i.at[...]], sem).wait()

    def _process(di, d2i, a_r, b_r, lo, kb=_KB):
        for j in range(kb):
            for k in range(_IB // 16):
                d = di[pl.ds(j * _IB + k * 16, 16)] - lo
                dummy = _HR + (k % 8) * 16 + lax.iota(jnp.int32, 16)
                inr = (d >= 0) & (d < _HR)
                d2i[pl.ds(j * _IB + k * 16, 16)] = jnp.where(inr, d, dummy)

        @plsc.parallel_loop(0, kb * _IB, unroll=8)
        def _edge(e):
            a_r[e] = jnp.maximum(a_r[e] + b_r[e], 0.0)

    def _zero_fill(i, carry):
        a0[i] = jnp.zeros((HID,), jnp.float32)
        return carry

    def _zero_slice(cnt):
        nfull = cnt // _EPC
        for k in range(nfull):
            pltpu.sync_copy(a0.at[...],
                            acc.at[pl.ds(r0 + k * _EPC, _EPC)])
        rem = cnt - nfull * _EPC
        if rem:
            pltpu.sync_copy(a0.at[pl.ds(0, rem)],
                            acc.at[pl.ds(r0 + nfull * _EPC, rem)])

    for p in range(2):
        lo = p * _HR
        lax.fori_loop(0, _EPC, _zero_fill, 0)

        @pl.when(s < _NS - 1)
        def _():
            _zero_slice(_RPTA)

        @pl.when(s == _NS - 1)
        def _():
            _zero_slice(_RPTL)

        plsc.subcore_barrier()

        @pl.when(_valid(0))
        def _():
            _fire_idx(0, si0, di0, sem_i0)
            _wait_idx(0, si0, di0, sem_i0)
            _fire_gathers(si0, di0, a0, b0, sem_g0)

        @pl.when(_valid(1))
        def _():
            _fire_idx(1, si1, di1, sem_i1)

        def _pair(u, carry, lo=lo):
            t0 = 2 * u
            t1 = 2 * u + 1

            @pl.when(_valid(t0))
            def _():
                _wait_gathers(si0, di0, a0, b0, sem_g0)

            @pl.when((u > 0) & _valid(t1 - 2))
            def _():
                _wait_scatters(a1, d2i1, sem_s1)

            @pl.when(_valid(t1))
            def _():
                _wait_idx(t1, si1, di1, sem_i1)
                _fire_gathers(si1, di1, a1, b1, sem_g1)

            @pl.when(_valid(t0))
            def _():
                _process(di0, d2i0, a0, b0, lo)
                _fire_scatters(a0, d2i0, sem_s0)

            @pl.when(_valid(t0 + 2))
            def _():
                _fire_idx(t0 + 2, si0, di0, sem_i0)

            @pl.when(_valid(t1))
            def _():
                _wait_gathers(si1, di1, a1, b1, sem_g1)

            @pl.when(_valid(t0 + 2))
            def _():
                _wait_scatters(a0, d2i0, sem_s0)
                _wait_idx(t0 + 2, si0, di0, sem_i0)
                _fire_gathers(si0, di0, a0, b0, sem_g0)

            @pl.when(_valid(t1))
            def _():
                _process(di1, d2i1, a1, b1, lo)
                _fire_scatters(a1, d2i1, sem_s1)

            @pl.when(_valid(t1 + 2))
            def _():
                _fire_idx(t1 + 2, si1, di1, sem_i1)

            return carry

        lax.fori_loop(0, _NPAIR, _pair, 0)

        _wait_scatters(a0, d2i0, sem_s0)

        if _TKB:
            @pl.when(wid == 0)
            def _tail():
                te0 = _NCH * _EPC
                tne = _TKB * _IB
                pltpu.sync_copy(ei_hbm.at[0, pl.ds(te0, tne)],
                                si0.at[pl.ds(0, tne)])
                pltpu.sync_copy(ei_hbm.at[1, pl.ds(te0, tne)],
                                di0.at[pl.ds(0, tne)])
                cps = [pltpu.async_copy(
                           a_hbm.at[si0.at[pl.ds(0, tne)]],
                           a0.at[pl.ds(0, tne)], sem_g0),
                       pltpu.async_copy(
                           b_hbm.at[di0.at[pl.ds(0, tne)]],
                           b0.at[pl.ds(0, tne)], sem_g0)]
                for cp in cps:
                    cp.wait()
                _process(di0, d2i0, a0, b0, lo, kb=_TKB)
                pltpu.async_copy(a0.at[pl.ds(0, tne)],
                                 acc.at[d2i0.at[pl.ds(0, tne)]],
                                 sem_s0, add=True).wait()

        plsc.subcore_barrier()

        @pl.when(s < _NS - 1)
        def _():
            pltpu.sync_copy(acc.at[pl.ds(r0, _RPTA)],
                            out_hbm.at[p, c, pl.ds(r0, _RPTA)])

        @pl.when(s == _NS - 1)
        def _():
            pltpu.sync_copy(acc.at[pl.ds(r0, _RPTL)],
                            out_hbm.at[p, c, pl.ds(r0, _RPTL)])

        plsc.subcore_barrier()


_sc_edge = functools.partial(
    pl.kernel,
    out_type=jax.ShapeDtypeStruct((2, _NC, _HR, HID), jnp.float32),
    mesh=plsc.VectorSubcoreMesh(core_axis_name="c", subcore_axis_name="s"),
    compiler_params=pltpu.CompilerParams(use_tc_tiling_on_sc=False),
    scratch_types=[
        pltpu.VMEM((_EPC,), jnp.int32),
        pltpu.VMEM((_EPC,), jnp.int32),
        pltpu.VMEM((_EPC,), jnp.int32),
        pltpu.VMEM((_EPC,), jnp.int32),
        pltpu.VMEM((_EPC,), jnp.int32),
        pltpu.VMEM((_EPC,), jnp.int32),
        pltpu.VMEM((_EPC, HID), jnp.float32),
        pltpu.VMEM((_EPC, HID), jnp.float32),
        pltpu.VMEM((_EPC, HID), jnp.float32),
        pltpu.VMEM((_EPC, HID), jnp.float32),
        pltpu.VMEM_SHARED((_HR + 128, HID), jnp.float32),
        pltpu.SemaphoreType.DMA,
        pltpu.SemaphoreType.DMA,
        pltpu.SemaphoreType.DMA,
        pltpu.SemaphoreType.DMA,
        pltpu.SemaphoreType.DMA,
        pltpu.SemaphoreType.DMA,
    ],
)(_sc_body)


_PB = _HR // _BLK


def _decode_body(outp_ref, acc_ref, wnd2_ref, out_ref):
    agg = acc_ref[0, 0, 0] + acc_ref[0, 1, 0]
    vel = jnp.dot(agg, wnd2_ref[...], preferred_element_type=jnp.float32)
    out_ref[...] = outp_ref[...] + vel.reshape(1, _RB, 32)


_decode = pl.pallas_call(
    _decode_body,
    grid=(_NBLK,),
    in_specs=[
        pl.BlockSpec((1, _RB, 32), lambda i: (i, 0, 0)),
        pl.BlockSpec((1, _NC, 1, _RB, 128),
                     lambda i: (i // _PB, 0, i % _PB, 0, 0)),
        pl.BlockSpec((128, 32), lambda i: (0, 0)),
    ],
    out_specs=pl.BlockSpec((1, _RB, 32), lambda i: (i, 0, 0)),
    out_shape=jax.ShapeDtypeStruct((_NBLK, _RB, 32), jnp.float32),
)


def kernel(t, node_f, edge_index, xf, W_ode, b_ode, W_ne, b_ne,
           W_ee, b_ee, W_nd, b_nd):
    del t
    wnd_pad = jnp.pad(W_nd, ((0, 0), (MD - 2, 0)))
    bode_eff = b_ode + jnp.pad(b_nd, (MD - 2, 0))
    eye8 = jnp.eye(8, dtype=jnp.float32)
    wne_bd = jnp.kron(eye8, W_ne)
    wtop_bd = jnp.kron(eye8, W_ee[:HID])
    wbot_bd = jnp.kron(eye8, W_ee[HID:])
    wnd1_bd = jnp.kron(eye8, wnd_pad[:HID])
    wnd2_bd = jnp.kron(eye8, wnd_pad[HID:])
    wode_bd = jnp.kron(
        eye8, jnp.concatenate([jnp.zeros((2, MD), jnp.float32), W_ode]))
    bne_t = jnp.tile(b_ne, 8).reshape(1, 128)
    bee_t = jnp.tile(b_ee, 8).reshape(1, 128)
    bode_t = jnp.tile(bode_eff, 8).reshape(1, 32)
    nfp = jnp.concatenate([xf, node_f], axis=1).reshape(_NBLK, _RB, 48)
    A2, B2, outp = _encode(nfp, wne_bd, bne_t, wtop_bd, wbot_bd, bee_t,
                           wode_bd, bode_t, wnd1_bd)
    acc4 = _sc_edge(edge_index, A2.reshape(N, HID), B2.reshape(N, HID))
    acc4r = acc4.reshape(2, _NC, _PB, _RB, 128)
    outf = _decode(outp, acc4r, wnd2_bd)
    return outf.reshape(N, MD)
